# baseline (speedup 1.0000x reference)
import jax
import jax.numpy as jnp
from jax import lax
from jax.experimental import pallas as pl
from jax.experimental.pallas import tpu as pltpu
from jax.experimental.pallas import tpu_sc as plsc

N = 10000
E = 160000
D = 256
DH = 128
L = 4
TYPE_VOCAB = 100
N_CLASSES = 250

NC = 2
NS = 16
NW = NC * NS

CHUNK = 80
NCHUNK = E // CHUNK
GCHUNK = 80
NGCHUNK = N // GCHUNK
RPT = 624
TAIL = N - NS * RPT
CP = 208
ROWS = 1000


CKD = 128
NCHD = E // CKD
DSLOT = 3


def _sc_deg_embed_body(ei_hbm, tok_hbm, tokemb_hbm, ones_hbm,
                       zeros_hbm, degp_hbm, h0b_hbm,
                       eidx, ones_v, nidx, rows_v, dstage,
                       dacc_out, dacc_in,
                       sem_i, sem_so, sem_sd, sem_tg, sem_tw):
    c = lax.axis_index("c")
    s = lax.axis_index("s")
    w = s * NC + c

    pltpu.sync_copy(ones_hbm, ones_v)

    @pl.when(s == 0)
    def _():
        pltpu.sync_copy(zeros_hbm, dacc_out)
        pltpu.sync_copy(zeros_hbm, dacc_in)

    plsc.subcore_barrier()

    mykd = (NCHD - w + NW - 1) // NW

    def wait_sc(sl):
        pltpu.make_async_copy(ones_hbm, ones_v, sem_so.at[sl]).wait()
        pltpu.make_async_copy(ones_hbm, ones_v, sem_sd.at[sl]).wait()

    def deg_step(kk, carry):
        for p in range(DSLOT):
            jf = DSLOT * kk + p
            sg = (p + DSLOT - 1) % DSLOT
            jg = jf - 1

            @pl.when(jnp.logical_and(jf >= DSLOT, jf < mykd))
            def _():
                wait_sc(p)

            @pl.when(jf < mykd)
            def _():
                cid = w + NW * jf
                pltpu.async_copy(ei_hbm.at[:, pl.ds(cid * CKD, CKD)],
                                 eidx.at[p], sem_i.at[p])

            @pl.when(jnp.logical_and(jg >= 0, jg < mykd))
            def _():
                pltpu.make_async_copy(ei_hbm.at[:, pl.ds(0, CKD)],
                                      eidx.at[sg], sem_i.at[sg]).wait()
                pltpu.async_copy(ones_v, dacc_out.at[eidx.at[sg, 0]],
                                 sem_so.at[sg], add=True)
                pltpu.async_copy(ones_v, dacc_in.at[eidx.at[sg, 1]],
                                 sem_sd.at[sg], add=True)
        return carry

    lax.fori_loop(0, (NCHD // NW + 2) // DSLOT + 1, deg_step, 0)

    for sl in range(DSLOT):
        jlast = (mykd - 1) - lax.rem(mykd - 1 - sl, DSLOT)

        @pl.when(jnp.logical_and(jlast >= 0, jlast >= mykd - DSLOT))
        def _():
            wait_sc(sl)

    mykg = (NGCHUNK - w + NW - 1) // NW

    def gat_step(k, carry):
        sl = lax.rem(k, 2)
        for q in range(2):
            @pl.when(jnp.logical_and(sl == q, k < mykg))
            def _():
                cid = w + NW * k

                @pl.when(k >= 2)
                def _():
                    pltpu.make_async_copy(h0b_hbm.at[pl.ds(0, GCHUNK)],
                                          rows_v.at[q], sem_tw.at[q]).wait()
                pltpu.sync_copy(tok_hbm.at[pl.ds(cid * GCHUNK, GCHUNK)],
                                nidx.at[q])
                pltpu.async_copy(tokemb_hbm.at[nidx.at[q]], rows_v.at[q],
                                 sem_tg.at[q]).wait()
                pltpu.async_copy(rows_v.at[q],
                                 h0b_hbm.at[pl.ds(cid * GCHUNK, GCHUNK)],
                                 sem_tw.at[q])
        return carry

    lax.fori_loop(0, (NGCHUNK + NW - 1) // NW, gat_step, 0)

    for q in range(2):
        @pl.when(mykg > q)
        def _():
            pltpu.make_async_copy(h0b_hbm.at[pl.ds(0, GCHUNK)],
                                  rows_v.at[q], sem_tw.at[q]).wait()

    plsc.subcore_barrier()

    @pl.when(s == 0)
    def _():
        pltpu.sync_copy(dacc_out, dstage)
        pltpu.sync_copy(dstage, degp_hbm.at[pl.ds(2 * c * N, N)])
        pltpu.sync_copy(dacc_in, dstage)
        pltpu.sync_copy(dstage, degp_hbm.at[pl.ds((2 * c + 1) * N, N)])


def _sc_deg_embed(edge_index, node_token, token_emb):
    mesh = plsc.VectorSubcoreMesh(core_axis_name="c", subcore_axis_name="s")
    f = pl.kernel(
        _sc_deg_embed_body,
        out_type=[
            jax.ShapeDtypeStruct((4 * N,), jnp.float32),
            jax.ShapeDtypeStruct((N, DH), jnp.float32),
        ],
        mesh=mesh,
        scratch_types=[
            pltpu.VMEM((DSLOT, 2, CKD), jnp.int32),
            pltpu.VMEM((CKD,), jnp.float32),
            pltpu.VMEM((2, GCHUNK), jnp.int32),
            pltpu.VMEM((2, GCHUNK, DH), jnp.float32),
            pltpu.VMEM((N,), jnp.float32),
            pltpu.VMEM_SHARED((N,), jnp.float32),
            pltpu.VMEM_SHARED((N,), jnp.float32),
            pltpu.SemaphoreType.DMA((DSLOT,)),
            pltpu.SemaphoreType.DMA((DSLOT,)),
            pltpu.SemaphoreType.DMA((DSLOT,)),
            pltpu.SemaphoreType.DMA((2,)),
            pltpu.SemaphoreType.DMA((2,)),
        ],
    )
    ones = jnp.ones((CKD,), jnp.float32)
    zeros = jnp.zeros((N,), jnp.float32)
    return f(edge_index, node_token, token_emb, ones, zeros)


NSLOT = 4
MAXK = NCHUNK // NS
NITER = (MAXK + 1) // NSLOT + 1


def _sc_agg_body(feat_hbm, ei_hbm, zeros_hbm, agg_hbm,
                 eidx, rows_v, acc, sem_i, sem_g, sem_s):
    c = lax.axis_index("c")
    s = lax.axis_index("s")

    for j in range(RPT // CP):
        pltpu.sync_copy(zeros_hbm, acc.at[pl.ds(s * RPT + j * CP, CP)])

    @pl.when(s == NS - 1)
    def _():
        pltpu.sync_copy(zeros_hbm.at[pl.ds(0, TAIL)], acc.at[pl.ds(NS * RPT, TAIL)])

    plsc.subcore_barrier()

    def wait_scatter(sl):
        pltpu.make_async_copy(feat_hbm.at[c].at[pl.ds(0, CHUNK)],
                              rows_v.at[sl], sem_s.at[sl]).wait()

    def step(kk, carry):
        for p in range(NSLOT):
            jf = NSLOT * kk + p
            sf = p
            sg = (p + NSLOT - 1) % NSLOT
            ss = (p + NSLOT - 2) % NSLOT
            jg = jf - 1
            js = jf - 2

            @pl.when(jnp.logical_and(jf >= NSLOT, jf < MAXK))
            def _():
                wait_scatter(sf)

            @pl.when(jf < MAXK)
            def _():
                cid = s + NS * jf
                pltpu.async_copy(ei_hbm.at[cid], eidx.at[sf], sem_i.at[sf])

            @pl.when(jnp.logical_and(jg >= 0, jg < MAXK))
            def _():
                pltpu.make_async_copy(ei_hbm.at[0], eidx.at[sg],
                                      sem_i.at[sg]).wait()
                pltpu.async_copy(feat_hbm.at[c].at[eidx.at[sg, 0]],
                                 rows_v.at[sg], sem_g.at[sg])

            @pl.when(jnp.logical_and(js >= 0, js < MAXK))
            def _():
                pltpu.make_async_copy(feat_hbm.at[c].at[pl.ds(0, CHUNK)],
                                      rows_v.at[ss], sem_g.at[ss]).wait()
                pltpu.async_copy(rows_v.at[ss], acc.at[eidx.at[ss, 1]],
                                 sem_s.at[ss], add=True)
        return carry

    lax.fori_loop(0, NITER, step, 0)

    for sl in range(NSLOT):
        jlast = (MAXK - 1) - ((MAXK - 1 - sl) % NSLOT)
        if jlast >= 0 and jlast >= MAXK - NSLOT:
            wait_scatter(sl)

    plsc.subcore_barrier()

    for j in range(RPT // CP):
        r = s * RPT + j * CP
        pltpu.sync_copy(acc.at[pl.ds(r, CP)], agg_hbm.at[c].at[pl.ds(r, CP)])

    @pl.when(s == NS - 1)
    def _():
        r = NS * RPT
        pltpu.sync_copy(acc.at[pl.ds(r, TAIL)], agg_hbm.at[c].at[pl.ds(r, TAIL)])


def _sc_agg(feat, ei3):
    mesh = plsc.VectorSubcoreMesh(core_axis_name="c", subcore_axis_name="s")
    f = pl.kernel(
        _sc_agg_body,
        out_type=jax.ShapeDtypeStruct((NC, N, DH), jnp.float32),
        mesh=mesh,
        scratch_types=[
            pltpu.VMEM((NSLOT, 2, CHUNK), jnp.int32),
            pltpu.VMEM((NSLOT, CHUNK, DH), jnp.float32),
            pltpu.VMEM_SHARED((N, DH), jnp.float32),
            pltpu.SemaphoreType.DMA((NSLOT,)),
            pltpu.SemaphoreType.DMA((NSLOT,)),
            pltpu.SemaphoreType.DMA((NSLOT,)),
        ],
    )
    zeros = jnp.zeros((CP, DH), jnp.float32)
    return f(feat, ei3, zeros)


def _prep_body(degp_ref, nt_ref, temb_ref, h0b_ref, ns_ref, nd_ref, feat_ref):
    d = degp_ref[...]
    deg_out = d[:, 0:1] + d[:, 2:3]
    deg_in = d[:, 1:2] + d[:, 3:4]
    ns = lax.rsqrt(jnp.maximum(deg_out, 1.0))
    nd = lax.rsqrt(jnp.maximum(deg_in, 1.0))
    ns_ref[...] = ns
    nd_ref[...] = nd
    nt = nt_ref[...]
    iota = lax.broadcasted_iota(jnp.int32, (ROWS, TYPE_VOCAB), 1)
    onehot = (nt == iota).astype(jnp.float32)
    typ = jnp.dot(onehot, temb_ref[...], preferred_element_type=jnp.float32,
                  precision=lax.Precision.HIGHEST)
    feat_ref[0] = typ * ns
    feat_ref[1] = h0b_ref[...] * ns


def _tc_prep(degp, node_type, type_emb, h0b):
    return pl.pallas_call(
        _prep_body,
        grid=(N // ROWS,),
        in_specs=[
            pl.BlockSpec((ROWS, 4), lambda i: (i, 0)),
            pl.BlockSpec((ROWS, 1), lambda i: (i, 0)),
            pl.BlockSpec((TYPE_VOCAB, DH), lambda i: (0, 0)),
            pl.BlockSpec((ROWS, DH), lambda i: (i, 0)),
        ],
        out_specs=[
            pl.BlockSpec((ROWS, 1), lambda i: (i, 0)),
            pl.BlockSpec((ROWS, 1), lambda i: (i, 0)),
            pl.BlockSpec((NC, ROWS, DH), lambda i: (0, i, 0)),
        ],
        out_shape=[
            jax.ShapeDtypeStruct((N, 1), jnp.float32),
            jax.ShapeDtypeStruct((N, 1), jnp.float32),
            jax.ShapeDtypeStruct((NC, N, DH), jnp.float32),
        ],
    )(degp, node_type, type_emb, h0b)


def _layer_body(agg_ref, nd_ref, ns_ref, w_ref, b_ref, out_ref):
    a = agg_ref[...]
    x = jnp.concatenate([a[0], a[1]], axis=1) * nd_ref[...]
    y = jnp.maximum(jnp.dot(x, w_ref[...], preferred_element_type=jnp.float32)
                    + b_ref[...], 0.0)
    f = y * ns_ref[...]
    out_ref[0] = f[:, :DH]
    out_ref[1] = f[:, DH:]


def _tc_layer(agg, norm_dst, norm_src, w, b):
    return pl.pallas_call(
        _layer_body,
        grid=(N // ROWS,),
        in_specs=[
            pl.BlockSpec((NC, ROWS, DH), lambda i: (0, i, 0)),
            pl.BlockSpec((ROWS, 1), lambda i: (i, 0)),
            pl.BlockSpec((ROWS, 1), lambda i: (i, 0)),
            pl.BlockSpec((D, D), lambda i: (0, 0)),
            pl.BlockSpec((1, D), lambda i: (0, 0)),
        ],
        out_specs=pl.BlockSpec((NC, ROWS, DH), lambda i: (0, i, 0)),
        out_shape=jax.ShapeDtypeStruct((NC, N, DH), jnp.float32),
    )(agg, norm_dst, norm_src, w, b[None, :])


def _layer_last_body(agg_ref, nd_ref, w_ref, b_ref, gw_ref, gb_ref,
                     h_ref, z_ref, m_ref):
    i = pl.program_id(0)
    a = agg_ref[...]
    x = jnp.concatenate([a[0], a[1]], axis=1) * nd_ref[...]
    y = jnp.maximum(jnp.dot(x, w_ref[...], preferred_element_type=jnp.float32)
                    + b_ref[...], 0.0)
    h_ref[...] = y
    z = jnp.dot(y, gw_ref[...], preferred_element_type=jnp.float32) + gb_ref[...]
    z_ref[...] = z

    @pl.when(i == 0)
    def _():
        m_ref[...] = jnp.full((1, 1), -jnp.inf, jnp.float32)
    m_ref[...] = jnp.maximum(m_ref[...], jnp.max(z))


def _tc_layer_last(agg, norm_dst, w, b, gate_w, gate_b):
    return pl.pallas_call(
        _layer_last_body,
        grid=(N // ROWS,),
        in_specs=[
            pl.BlockSpec((NC, ROWS, DH), lambda i: (0, i, 0)),
            pl.BlockSpec((ROWS, 1), lambda i: (i, 0)),
            pl.BlockSpec((D, D), lambda i: (0, 0)),
            pl.BlockSpec((1, D), lambda i: (0, 0)),
            pl.BlockSpec((D, 1), lambda i: (0, 0)),
            pl.BlockSpec((1, 1), lambda i: (0, 0)),
        ],
        out_specs=[
            pl.BlockSpec((ROWS, D), lambda i: (i, 0)),
            pl.BlockSpec((ROWS, 1), lambda i: (i, 0)),
            pl.BlockSpec((1, 1), lambda i: (0, 0)),
        ],
        out_shape=[
            jax.ShapeDtypeStruct((N, D), jnp.float32),
            jax.ShapeDtypeStruct((N, 1), jnp.float32),
            jax.ShapeDtypeStruct((1, 1), jnp.float32),
        ],
    )(agg, norm_dst, w, b[None, :], gate_w, gate_b[None, :])


def _pool_body(h_ref, z_ref, m_ref, cw_ref, cb_ref, out_ref, s_ref, r_ref):
    i = pl.program_id(0)
    nblk = pl.num_programs(0)

    @pl.when(i == 0)
    def _():
        s_ref[0, 0] = 0.0
        r_ref[...] = jnp.zeros_like(r_ref)
    e = jnp.exp(z_ref[...] - m_ref[...])
    s_ref[0, 0] += jnp.sum(e)
    r_ref[...] += jnp.dot(e.T, h_ref[...], preferred_element_type=jnp.float32)

    @pl.when(i == nblk - 1)
    def _():
        readout = r_ref[...] / s_ref[0, 0]
        out_ref[...] = jnp.dot(readout, cw_ref[...],
                               preferred_element_type=jnp.float32) + cb_ref[...]


def _tc_pool(h, z, m, cls_w, cls_b):
    cls_w_p = jnp.zeros((D, 256), jnp.float32).at[:, :N_CLASSES].set(cls_w)
    cls_b_p = jnp.zeros((1, 256), jnp.float32).at[0, :N_CLASSES].set(cls_b)
    out = pl.pallas_call(
        _pool_body,
        grid=(N // ROWS,),
        in_specs=[
            pl.BlockSpec((ROWS, D), lambda i: (i, 0)),
            pl.BlockSpec((ROWS, 1), lambda i: (i, 0)),
            pl.BlockSpec((1, 1), lambda i: (0, 0)),
            pl.BlockSpec((D, 256), lambda i: (0, 0)),
            pl.BlockSpec((1, 256), lambda i: (0, 0)),
        ],
        out_specs=pl.BlockSpec((1, 256), lambda i: (0, 0)),
        out_shape=jax.ShapeDtypeStruct((1, 256), jnp.float32),
        scratch_shapes=[
            pltpu.SMEM((1, 1), jnp.float32),
            pltpu.VMEM((1, D), jnp.float32),
        ],
    )(h, z, m, cls_w_p, cls_b_p)
    return out[:, :N_CLASSES]


def kernel(node_type, node_token, edge_index, type_emb, token_emb, W, b,
           gate_W, gate_b, cls_W, cls_b):
    ei3 = edge_index.reshape(2, NCHUNK, CHUNK).swapaxes(0, 1)
    degp, h0b = _sc_deg_embed(edge_index, node_token, token_emb)
    norm_src, norm_dst, feat = _tc_prep(degp.reshape(4, N).T,
                                        node_type[:, None], type_emb, h0b)
    for i in range(L - 1):
        agg = _sc_agg(feat, ei3)
        feat = _tc_layer(agg, norm_dst, norm_src, W[i], b[i])
    agg = _sc_agg(feat, ei3)
    h, z, m = _tc_layer_last(agg, norm_dst, W[L - 1], b[L - 1], gate_W, gate_b)
    return _tc_pool(h, z, m, cls_W, cls_b)


# speedup vs baseline: 1.0210x; 1.0210x over previous
import jax
import jax.numpy as jnp
from jax import lax
from jax.experimental import pallas as pl
from jax.experimental.pallas import tpu as pltpu
from jax.experimental.pallas import tpu_sc as plsc

N = 10000
E = 160000
D = 256
DH = 128
L = 4
TYPE_VOCAB = 100
N_CLASSES = 250

NC = 2
NS = 16
NW = NC * NS

CHUNK = 80
NCHUNK = E // CHUNK
GCHUNK = 80
NGCHUNK = N // GCHUNK
RPT = 624
TAIL = N - NS * RPT
CP = 208
ROWS = 2000


CKD = 128
NCHD = E // CKD
DSLOT = 3


def _sc_deg_embed_body(ei_hbm, tok_hbm, tokemb_hbm, ones_hbm,
                       zeros_hbm, degp_hbm, h0b_hbm,
                       eidx, ones_v, nidx, rows_v, dstage,
                       dacc_out, dacc_in,
                       sem_i, sem_so, sem_sd, sem_tg, sem_tw):
    c = lax.axis_index("c")
    s = lax.axis_index("s")
    w = s * NC + c

    pltpu.sync_copy(ones_hbm, ones_v)

    @pl.when(s == 0)
    def _():
        pltpu.sync_copy(zeros_hbm, dacc_out)
        pltpu.sync_copy(zeros_hbm, dacc_in)

    plsc.subcore_barrier()

    mykd = (NCHD - w + NW - 1) // NW

    def wait_sc(sl):
        pltpu.make_async_copy(ones_hbm, ones_v, sem_so.at[sl]).wait()
        pltpu.make_async_copy(ones_hbm, ones_v, sem_sd.at[sl]).wait()

    def deg_step(kk, carry):
        for p in range(DSLOT):
            jf = DSLOT * kk + p
            sg = (p + DSLOT - 1) % DSLOT
            jg = jf - 1

            @pl.when(jnp.logical_and(jf >= DSLOT, jf < mykd))
            def _():
                wait_sc(p)

            @pl.when(jf < mykd)
            def _():
                cid = w + NW * jf
                pltpu.async_copy(ei_hbm.at[:, pl.ds(cid * CKD, CKD)],
                                 eidx.at[p], sem_i.at[p])

            @pl.when(jnp.logical_and(jg >= 0, jg < mykd))
            def _():
                pltpu.make_async_copy(ei_hbm.at[:, pl.ds(0, CKD)],
                                      eidx.at[sg], sem_i.at[sg]).wait()
                pltpu.async_copy(ones_v, dacc_out.at[eidx.at[sg, 0]],
                                 sem_so.at[sg], add=True)
                pltpu.async_copy(ones_v, dacc_in.at[eidx.at[sg, 1]],
                                 sem_sd.at[sg], add=True)
        return carry

    lax.fori_loop(0, (NCHD // NW + 2) // DSLOT + 1, deg_step, 0)

    for sl in range(DSLOT):
        jlast = (mykd - 1) - lax.rem(mykd - 1 - sl, DSLOT)

        @pl.when(jnp.logical_and(jlast >= 0, jlast >= mykd - DSLOT))
        def _():
            wait_sc(sl)

    mykg = (NGCHUNK - w + NW - 1) // NW

    def gat_step(k, carry):
        sl = lax.rem(k, 2)
        for q in range(2):
            @pl.when(jnp.logical_and(sl == q, k < mykg))
            def _():
                cid = w + NW * k

                @pl.when(k >= 2)
                def _():
                    pltpu.make_async_copy(h0b_hbm.at[pl.ds(0, GCHUNK)],
                                          rows_v.at[q], sem_tw.at[q]).wait()
                pltpu.sync_copy(tok_hbm.at[pl.ds(cid * GCHUNK, GCHUNK)],
                                nidx.at[q])
                pltpu.async_copy(tokemb_hbm.at[nidx.at[q]], rows_v.at[q],
                                 sem_tg.at[q]).wait()
                pltpu.async_copy(rows_v.at[q],
                                 h0b_hbm.at[pl.ds(cid * GCHUNK, GCHUNK)],
                                 sem_tw.at[q])
        return carry

    lax.fori_loop(0, (NGCHUNK + NW - 1) // NW, gat_step, 0)

    for q in range(2):
        @pl.when(mykg > q)
        def _():
            pltpu.make_async_copy(h0b_hbm.at[pl.ds(0, GCHUNK)],
                                  rows_v.at[q], sem_tw.at[q]).wait()

    plsc.subcore_barrier()

    @pl.when(s == 0)
    def _():
        pltpu.sync_copy(dacc_out, dstage)
        pltpu.sync_copy(dstage, degp_hbm.at[pl.ds(2 * c * N, N)])
        pltpu.sync_copy(dacc_in, dstage)
        pltpu.sync_copy(dstage, degp_hbm.at[pl.ds((2 * c + 1) * N, N)])


def _sc_deg_embed(edge_index, node_token, token_emb):
    mesh = plsc.VectorSubcoreMesh(core_axis_name="c", subcore_axis_name="s")
    f = pl.kernel(
        _sc_deg_embed_body,
        out_type=[
            jax.ShapeDtypeStruct((4 * N,), jnp.float32),
            jax.ShapeDtypeStruct((N, DH), jnp.float32),
        ],
        mesh=mesh,
        scratch_types=[
            pltpu.VMEM((DSLOT, 2, CKD), jnp.int32),
            pltpu.VMEM((CKD,), jnp.float32),
            pltpu.VMEM((2, GCHUNK), jnp.int32),
            pltpu.VMEM((2, GCHUNK, DH), jnp.float32),
            pltpu.VMEM((N,), jnp.float32),
            pltpu.VMEM_SHARED((N,), jnp.float32),
            pltpu.VMEM_SHARED((N,), jnp.float32),
            pltpu.SemaphoreType.DMA((DSLOT,)),
            pltpu.SemaphoreType.DMA((DSLOT,)),
            pltpu.SemaphoreType.DMA((DSLOT,)),
            pltpu.SemaphoreType.DMA((2,)),
            pltpu.SemaphoreType.DMA((2,)),
        ],
    )
    ones = jnp.ones((CKD,), jnp.float32)
    zeros = jnp.zeros((N,), jnp.float32)
    return f(edge_index, node_token, token_emb, ones, zeros)


NSLOT = 4
MAXK = NCHUNK // NS
NITER = (MAXK + 1) // NSLOT + 1


def _sc_agg_body(feat_hbm, ei_hbm, zeros_hbm, agg_hbm,
                 eidx, rows_v, acc, sem_i, sem_g, sem_s):
    c = lax.axis_index("c")
    s = lax.axis_index("s")

    for j in range(RPT // CP):
        pltpu.sync_copy(zeros_hbm, acc.at[pl.ds(s * RPT + j * CP, CP)])

    @pl.when(s == NS - 1)
    def _():
        pltpu.sync_copy(zeros_hbm.at[pl.ds(0, TAIL)], acc.at[pl.ds(NS * RPT, TAIL)])

    plsc.subcore_barrier()

    def wait_scatter(sl):
        pltpu.make_async_copy(feat_hbm.at[c].at[pl.ds(0, CHUNK)],
                              rows_v.at[sl], sem_s.at[sl]).wait()

    def step(kk, carry):
        for p in range(NSLOT):
            jf = NSLOT * kk + p
            sf = p
            sg = (p + NSLOT - 1) % NSLOT
            ss = (p + NSLOT - 2) % NSLOT
            jg = jf - 1
            js = jf - 2

            @pl.when(jnp.logical_and(jf >= NSLOT, jf < MAXK))
            def _():
                wait_scatter(sf)

            @pl.when(jf < MAXK)
            def _():
                cid = s + NS * jf
                pltpu.async_copy(ei_hbm.at[cid], eidx.at[sf], sem_i.at[sf])

            @pl.when(jnp.logical_and(jg >= 0, jg < MAXK))
            def _():
                pltpu.make_async_copy(ei_hbm.at[0], eidx.at[sg],
                                      sem_i.at[sg]).wait()
                pltpu.async_copy(feat_hbm.at[c].at[eidx.at[sg, 0]],
                                 rows_v.at[sg], sem_g.at[sg])

            @pl.when(jnp.logical_and(js >= 0, js < MAXK))
            def _():
                pltpu.make_async_copy(feat_hbm.at[c].at[pl.ds(0, CHUNK)],
                                      rows_v.at[ss], sem_g.at[ss]).wait()
                pltpu.async_copy(rows_v.at[ss], acc.at[eidx.at[ss, 1]],
                                 sem_s.at[ss], add=True)
        return carry

    lax.fori_loop(0, NITER, step, 0)

    for sl in range(NSLOT):
        jlast = (MAXK - 1) - ((MAXK - 1 - sl) % NSLOT)
        if jlast >= 0 and jlast >= MAXK - NSLOT:
            wait_scatter(sl)

    plsc.subcore_barrier()

    for j in range(RPT // CP):
        r = s * RPT + j * CP
        pltpu.sync_copy(acc.at[pl.ds(r, CP)], agg_hbm.at[c].at[pl.ds(r, CP)])

    @pl.when(s == NS - 1)
    def _():
        r = NS * RPT
        pltpu.sync_copy(acc.at[pl.ds(r, TAIL)], agg_hbm.at[c].at[pl.ds(r, TAIL)])


def _sc_agg(feat, ei3):
    mesh = plsc.VectorSubcoreMesh(core_axis_name="c", subcore_axis_name="s")
    f = pl.kernel(
        _sc_agg_body,
        out_type=jax.ShapeDtypeStruct((NC, N, DH), jnp.float32),
        mesh=mesh,
        scratch_types=[
            pltpu.VMEM((NSLOT, 2, CHUNK), jnp.int32),
            pltpu.VMEM((NSLOT, CHUNK, DH), jnp.float32),
            pltpu.VMEM_SHARED((N, DH), jnp.float32),
            pltpu.SemaphoreType.DMA((NSLOT,)),
            pltpu.SemaphoreType.DMA((NSLOT,)),
            pltpu.SemaphoreType.DMA((NSLOT,)),
        ],
    )
    zeros = jnp.zeros((CP, DH), jnp.float32)
    return f(feat, ei3, zeros)


def _prep_body(degp_ref, nt_ref, temb_ref, h0b_ref, ns_ref, nd_ref, feat_ref):
    d = degp_ref[...]
    deg_out = d[:, 0:1] + d[:, 2:3]
    deg_in = d[:, 1:2] + d[:, 3:4]
    ns = lax.rsqrt(jnp.maximum(deg_out, 1.0))
    nd = lax.rsqrt(jnp.maximum(deg_in, 1.0))
    ns_ref[...] = ns
    nd_ref[...] = nd
    nt = nt_ref[...]
    iota = lax.broadcasted_iota(jnp.int32, (ROWS, TYPE_VOCAB), 1)
    onehot = (nt == iota).astype(jnp.float32)
    typ = jnp.dot(onehot, temb_ref[...], preferred_element_type=jnp.float32,
                  precision=lax.Precision.HIGHEST)
    feat_ref[0] = typ * ns
    feat_ref[1] = h0b_ref[...] * ns


def _tc_prep(degp, node_type, type_emb, h0b):
    return pl.pallas_call(
        _prep_body,
        grid=(N // ROWS,),
        in_specs=[
            pl.BlockSpec((ROWS, 4), lambda i: (i, 0)),
            pl.BlockSpec((ROWS, 1), lambda i: (i, 0)),
            pl.BlockSpec((TYPE_VOCAB, DH), lambda i: (0, 0)),
            pl.BlockSpec((ROWS, DH), lambda i: (i, 0)),
        ],
        out_specs=[
            pl.BlockSpec((ROWS, 1), lambda i: (i, 0)),
            pl.BlockSpec((ROWS, 1), lambda i: (i, 0)),
            pl.BlockSpec((NC, ROWS, DH), lambda i: (0, i, 0)),
        ],
        out_shape=[
            jax.ShapeDtypeStruct((N, 1), jnp.float32),
            jax.ShapeDtypeStruct((N, 1), jnp.float32),
            jax.ShapeDtypeStruct((NC, N, DH), jnp.float32),
        ],
    )(degp, node_type, type_emb, h0b)


def _layer_body(agg_ref, nd_ref, ns_ref, w_ref, b_ref, out_ref):
    a = agg_ref[...]
    x = jnp.concatenate([a[0], a[1]], axis=1) * nd_ref[...]
    y = jnp.maximum(jnp.dot(x, w_ref[...], preferred_element_type=jnp.float32)
                    + b_ref[...], 0.0)
    f = y * ns_ref[...]
    out_ref[0] = f[:, :DH]
    out_ref[1] = f[:, DH:]


def _tc_layer(agg, norm_dst, norm_src, w, b):
    return pl.pallas_call(
        _layer_body,
        grid=(N // ROWS,),
        in_specs=[
            pl.BlockSpec((NC, ROWS, DH), lambda i: (0, i, 0)),
            pl.BlockSpec((ROWS, 1), lambda i: (i, 0)),
            pl.BlockSpec((ROWS, 1), lambda i: (i, 0)),
            pl.BlockSpec((D, D), lambda i: (0, 0)),
            pl.BlockSpec((1, D), lambda i: (0, 0)),
        ],
        out_specs=pl.BlockSpec((NC, ROWS, DH), lambda i: (0, i, 0)),
        out_shape=jax.ShapeDtypeStruct((NC, N, DH), jnp.float32),
    )(agg, norm_dst, norm_src, w, b[None, :])


def _layer_last_body(agg_ref, nd_ref, w_ref, b_ref, gw_ref, gb_ref,
                     h_ref, z_ref, m_ref):
    i = pl.program_id(0)
    a = agg_ref[...]
    x = jnp.concatenate([a[0], a[1]], axis=1) * nd_ref[...]
    y = jnp.maximum(jnp.dot(x, w_ref[...], preferred_element_type=jnp.float32)
                    + b_ref[...], 0.0)
    h_ref[...] = y
    z = jnp.dot(y, gw_ref[...], preferred_element_type=jnp.float32) + gb_ref[...]
    z_ref[...] = z

    @pl.when(i == 0)
    def _():
        m_ref[...] = jnp.full((1, 1), -jnp.inf, jnp.float32)
    m_ref[...] = jnp.maximum(m_ref[...], jnp.max(z))


def _tc_layer_last(agg, norm_dst, w, b, gate_w, gate_b):
    return pl.pallas_call(
        _layer_last_body,
        grid=(N // ROWS,),
        in_specs=[
            pl.BlockSpec((NC, ROWS, DH), lambda i: (0, i, 0)),
            pl.BlockSpec((ROWS, 1), lambda i: (i, 0)),
            pl.BlockSpec((D, D), lambda i: (0, 0)),
            pl.BlockSpec((1, D), lambda i: (0, 0)),
            pl.BlockSpec((D, 1), lambda i: (0, 0)),
            pl.BlockSpec((1, 1), lambda i: (0, 0)),
        ],
        out_specs=[
            pl.BlockSpec((ROWS, D), lambda i: (i, 0)),
            pl.BlockSpec((ROWS, 1), lambda i: (i, 0)),
            pl.BlockSpec((1, 1), lambda i: (0, 0)),
        ],
        out_shape=[
            jax.ShapeDtypeStruct((N, D), jnp.float32),
            jax.ShapeDtypeStruct((N, 1), jnp.float32),
            jax.ShapeDtypeStruct((1, 1), jnp.float32),
        ],
    )(agg, norm_dst, w, b[None, :], gate_w, gate_b[None, :])


def _pool_body(h_ref, z_ref, m_ref, cw_ref, cb_ref, out_ref, s_ref, r_ref):
    i = pl.program_id(0)
    nblk = pl.num_programs(0)

    @pl.when(i == 0)
    def _():
        s_ref[0, 0] = 0.0
        r_ref[...] = jnp.zeros_like(r_ref)
    e = jnp.exp(z_ref[...] - m_ref[...])
    s_ref[0, 0] += jnp.sum(e)
    r_ref[...] += jnp.dot(e.T, h_ref[...], preferred_element_type=jnp.float32)

    @pl.when(i == nblk - 1)
    def _():
        readout = r_ref[...] / s_ref[0, 0]
        out_ref[...] = jnp.dot(readout, cw_ref[...],
                               preferred_element_type=jnp.float32) + cb_ref[...]


def _tc_pool(h, z, m, cls_w, cls_b):
    cls_w_p = jnp.zeros((D, 256), jnp.float32).at[:, :N_CLASSES].set(cls_w)
    cls_b_p = jnp.zeros((1, 256), jnp.float32).at[0, :N_CLASSES].set(cls_b)
    out = pl.pallas_call(
        _pool_body,
        grid=(N // ROWS,),
        in_specs=[
            pl.BlockSpec((ROWS, D), lambda i: (i, 0)),
            pl.BlockSpec((ROWS, 1), lambda i: (i, 0)),
            pl.BlockSpec((1, 1), lambda i: (0, 0)),
            pl.BlockSpec((D, 256), lambda i: (0, 0)),
            pl.BlockSpec((1, 256), lambda i: (0, 0)),
        ],
        out_specs=pl.BlockSpec((1, 256), lambda i: (0, 0)),
        out_shape=jax.ShapeDtypeStruct((1, 256), jnp.float32),
        scratch_shapes=[
            pltpu.SMEM((1, 1), jnp.float32),
            pltpu.VMEM((1, D), jnp.float32),
        ],
    )(h, z, m, cls_w_p, cls_b_p)
    return out[:, :N_CLASSES]


def kernel(node_type, node_token, edge_index, type_emb, token_emb, W, b,
           gate_W, gate_b, cls_W, cls_b):
    ei3 = edge_index.reshape(2, NCHUNK, CHUNK).swapaxes(0, 1)
    degp, h0b = _sc_deg_embed(edge_index, node_token, token_emb)
    norm_src, norm_dst, feat = _tc_prep(degp.reshape(4, N).T,
                                        node_type[:, None], type_emb, h0b)
    for i in range(L - 1):
        agg = _sc_agg(feat, ei3)
        feat = _tc_layer(agg, norm_dst, norm_src, W[i], b[i])
    agg = _sc_agg(feat, ei3)
    h, z, m = _tc_layer_last(agg, norm_dst, W[L - 1], b[L - 1], gate_W, gate_b)
    return _tc_pool(h, z, m, cls_W, cls_b)
